# Initial kernel scaffold; baseline (speedup 1.0000x reference)
import jax
import jax.numpy as jnp
from jax.experimental import pallas as pl


def kernel(x, w_skip_mat, s_skip, b_skip, w1_mat, s1, b1, w2_col, s2, b2, w3_mat, s3, b3):
    raise NotImplementedError("write your pallas kernel here")


# speedup vs baseline: 1.3387x; 1.3387x over previous
import functools

import jax
import jax.numpy as jnp
from jax.experimental import pallas as pl
from jax.experimental.pallas import tpu as pltpu

_LEAKY = 0.01


def _block_body(H, W, D, x_m1_ref, x_0_ref, x_p1_ref, wsk_ref, ssk_ref,
                bsk_ref, w1_ref, s1_ref, b1_ref, w2_ref, s2_ref, b2_ref,
                w3_ref, s3_ref, b3_ref, o_ref, h1pad_ref, col_ref):
    d = pl.program_id(1)
    C, HW = x_0_ref.shape
    PADL = W + 1

    h1pad_ref[...] = jnp.zeros_like(h1pad_ref)
    for kd, xr in ((0, x_m1_ref), (1, x_0_ref), (2, x_p1_ref)):
        dz = d + kd - 1

        @pl.when(jnp.logical_and(dz >= 0, dz < D))
        def _(kd=kd, xr=xr):
            y = jnp.dot(w1_ref[...], xr[...],
                        preferred_element_type=jnp.float32)
            y = y * s1_ref[...] + b1_ref[...]
            y = jnp.where(y >= 0, y, _LEAKY * y)
            h1pad_ref[kd, :, PADL:PADL + HW] = y.astype(jnp.bfloat16)

    wpos = jax.lax.broadcasted_iota(jnp.int32, (C, HW), 1) % W
    mask_l = wpos != 0
    mask_r = wpos != (W - 1)
    for kd in range(3):
        for kh in range(3):
            for kw in range(3):
                j = (kd * 3 + kh) * 3 + kw
                t = h1pad_ref[kd, :, kh * W + kw:kh * W + kw + HW]
                if kw == 0:
                    t = jnp.where(mask_l, t, jnp.bfloat16(0))
                elif kw == 2:
                    t = jnp.where(mask_r, t, jnp.bfloat16(0))
                col_ref[j * C:(j + 1) * C, :] = t

    h2 = jnp.dot(w2_ref[...], col_ref[...], preferred_element_type=jnp.float32)
    h2 = h2 * s2_ref[...] + b2_ref[...]
    h2 = jnp.where(h2 >= 0, h2, _LEAKY * h2).astype(jnp.bfloat16)

    res = jnp.dot(wsk_ref[...], x_0_ref[...],
                  preferred_element_type=jnp.float32)
    res = res * ssk_ref[...] + bsk_ref[...]
    y = jnp.dot(w3_ref[...], h2, preferred_element_type=jnp.float32)
    y = y * s3_ref[...] + b3_ref[...] + res
    o_ref[...] = jnp.maximum(y, 0.0)


def kernel(x, w_skip_mat, s_skip, b_skip, w1_mat, s1, b1, w2_col, s2, b2,
           w3_mat, s3, b3):
    N, Cin, D, H, W = x.shape
    HW = H * W
    Cmid = w1_mat.shape[1]
    Cout = w_skip_mat.shape[1]
    PAD = HW + 2 * W + 2

    xb = x.reshape(N, Cin, D * HW).astype(jnp.bfloat16)
    wskT = w_skip_mat.T.astype(jnp.bfloat16)
    w1T = w1_mat.T.astype(jnp.bfloat16)
    w2m = w2_col.T.astype(jnp.bfloat16)
    w3T = w3_mat.T.astype(jnp.bfloat16)

    out = pl.pallas_call(
        functools.partial(_block_body, H, W, D),
        out_shape=jax.ShapeDtypeStruct((N, Cout, D * HW), jnp.float32),
        grid_spec=pltpu.PrefetchScalarGridSpec(
            num_scalar_prefetch=0,
            grid=(N, D),
            in_specs=[
                pl.BlockSpec((None, Cin, HW),
                             lambda n, d: (n, 0, jnp.maximum(d - 1, 0))),
                pl.BlockSpec((None, Cin, HW), lambda n, d: (n, 0, d)),
                pl.BlockSpec((None, Cin, HW),
                             lambda n, d: (n, 0, jnp.minimum(d + 1, D - 1))),
                pl.BlockSpec((Cout, Cin), lambda n, d: (0, 0)),
                pl.BlockSpec((Cout, 1), lambda n, d: (0, 0)),
                pl.BlockSpec((Cout, 1), lambda n, d: (0, 0)),
                pl.BlockSpec((Cmid, Cin), lambda n, d: (0, 0)),
                pl.BlockSpec((Cmid, 1), lambda n, d: (0, 0)),
                pl.BlockSpec((Cmid, 1), lambda n, d: (0, 0)),
                pl.BlockSpec((Cmid, 27 * Cmid), lambda n, d: (0, 0)),
                pl.BlockSpec((Cmid, 1), lambda n, d: (0, 0)),
                pl.BlockSpec((Cmid, 1), lambda n, d: (0, 0)),
                pl.BlockSpec((Cout, Cmid), lambda n, d: (0, 0)),
                pl.BlockSpec((Cout, 1), lambda n, d: (0, 0)),
                pl.BlockSpec((Cout, 1), lambda n, d: (0, 0)),
            ],
            out_specs=pl.BlockSpec((None, Cout, HW), lambda n, d: (n, 0, d)),
            scratch_shapes=[
                pltpu.VMEM((3, Cmid, PAD), jnp.bfloat16),
                pltpu.VMEM((27 * Cmid, HW), jnp.bfloat16),
            ],
        ),
        compiler_params=pltpu.CompilerParams(
            dimension_semantics=("parallel", "parallel"),
            vmem_limit_bytes=64 * 1024 * 1024),
    )(xb, xb, xb, wskT, s_skip.reshape(Cout, 1), b_skip.reshape(Cout, 1),
      w1T, s1.reshape(Cmid, 1), b1.reshape(Cmid, 1),
      w2m, s2.reshape(Cmid, 1), b2.reshape(Cmid, 1),
      w3T, s3.reshape(Cout, 1), b3.reshape(Cout, 1))
    return out.reshape(N, Cout, D, H, W)


# speedup vs baseline: 1.8193x; 1.3589x over previous
import functools

import jax
import jax.numpy as jnp
from jax.experimental import pallas as pl
from jax.experimental.pallas import tpu as pltpu

_LEAKY = 0.01


def _block_body(H, W, D2, x_m1_ref, x_0_ref, x_p1_ref, x_p2_ref, w1_ref,
                b1_ref, w2s_ref, b2_ref, wco_ref, bco_ref, o_ref, h1pad_ref):
    d2 = pl.program_id(1)
    C, HW = x_0_ref.shape
    PADL = 128
    PADT = h1pad_ref.shape[1]

    x4 = jnp.concatenate([x_m1_ref[...], x_0_ref[...], x_p1_ref[...],
                          x_p2_ref[...]], axis=1)
    y4 = jnp.dot(w1_ref[...], x4, preferred_element_type=jnp.float32)
    y4 = y4 + b1_ref[...]
    y4 = jnp.where(y4 >= 0, y4, _LEAKY * y4).astype(jnp.bfloat16)

    h1pad_ref[:, 0:PADL] = jnp.zeros((4 * C, PADL), jnp.bfloat16)
    h1pad_ref[:, PADL + HW:PADT] = jnp.zeros((4 * C, PADT - PADL - HW),
                                             jnp.bfloat16)
    for k in range(4):
        yk = y4[:, k * HW:(k + 1) * HW]
        if k == 0:
            yk = yk * (d2 > 0).astype(jnp.bfloat16)
        elif k == 3:
            yk = yk * (d2 < D2 - 1).astype(jnp.bfloat16)
        h1pad_ref[k * C:(k + 1) * C, PADL:PADL + HW] = yk

    wpos = jax.lax.broadcasted_iota(jnp.int32, (4 * C, HW), 1) % W
    mask_l = wpos != 0
    mask_r = wpos != (W - 1)
    acc = None
    for kh in range(3):
        for kw in range(3):
            j = kh * 3 + kw
            off = PADL + (kh - 1) * W + (kw - 1)
            r = h1pad_ref[:, off:off + HW]
            if kw == 0:
                r = jnp.where(mask_l, r, jnp.bfloat16(0))
            elif kw == 2:
                r = jnp.where(mask_r, r, jnp.bfloat16(0))
            w = w2s_ref[j * 2 * C:(j + 1) * 2 * C, :]
            t = jnp.dot(w, r, preferred_element_type=jnp.float32)
            acc = t if acc is None else acc + t
    h2 = acc + b2_ref[...]
    h2 = jnp.where(h2 >= 0, h2, _LEAKY * h2).astype(jnp.bfloat16)

    xh = jnp.concatenate([x_0_ref[...], x_p1_ref[...], h2], axis=0)
    y = jnp.dot(wco_ref[...], xh, preferred_element_type=jnp.float32)
    y = jnp.maximum(y + bco_ref[...], 0.0)
    Cout = y.shape[0] // 2
    o_ref[:, 0:HW] = y[0:Cout]
    o_ref[:, HW:2 * HW] = y[Cout:2 * Cout]


def kernel(x, w_skip_mat, s_skip, b_skip, w1_mat, s1, b1, w2_col, s2, b2,
           w3_mat, s3, b3):
    N, Cin, D, H, W = x.shape
    HW = H * W
    D2 = D // 2
    Cmid = w1_mat.shape[1]
    Cout = w_skip_mat.shape[1]
    PADT = 128 + HW + 128

    xb = x.reshape(N, Cin, D * HW).astype(jnp.bfloat16)

    w1s = (s1[:, None] * w1_mat.T).astype(jnp.bfloat16)

    w2t = w2_col.reshape(3, 3, 3, Cmid, Cmid)
    w2s = jnp.zeros((9, 2 * Cmid, 4 * Cmid), jnp.float32)
    for kh in range(3):
        for kw in range(3):
            j = kh * 3 + kw
            for kd in range(3):
                wt = s2[:, None] * w2t[kd, kh, kw].T
                w2s = w2s.at[j, 0:Cmid, kd * Cmid:(kd + 1) * Cmid].set(wt)
                w2s = w2s.at[j, Cmid:, (kd + 1) * Cmid:(kd + 2) * Cmid].set(wt)
    w2s = w2s.reshape(9 * 2 * Cmid, 4 * Cmid).astype(jnp.bfloat16)

    wsk = s_skip[:, None] * w_skip_mat.T
    w3s = s3[:, None] * w3_mat.T
    wco = jnp.zeros((2 * Cout, 2 * Cin + 2 * Cmid), jnp.float32)
    wco = wco.at[0:Cout, 0:Cin].set(wsk)
    wco = wco.at[0:Cout, 2 * Cin:2 * Cin + Cmid].set(w3s)
    wco = wco.at[Cout:, Cin:2 * Cin].set(wsk)
    wco = wco.at[Cout:, 2 * Cin + Cmid:].set(w3s)
    wco = wco.astype(jnp.bfloat16)

    b1c = b1.reshape(Cmid, 1)
    b2c = jnp.concatenate([b2, b2]).reshape(2 * Cmid, 1)
    bco = jnp.concatenate([b_skip + b3, b_skip + b3]).reshape(2 * Cout, 1)

    xspec = lambda f: pl.BlockSpec((None, Cin, HW), f)
    out = pl.pallas_call(
        functools.partial(_block_body, H, W, D2),
        out_shape=jax.ShapeDtypeStruct((N, Cout, D * HW), jnp.float32),
        grid_spec=pltpu.PrefetchScalarGridSpec(
            num_scalar_prefetch=0,
            grid=(N, D2),
            in_specs=[
                xspec(lambda n, d: (n, 0, jnp.maximum(2 * d - 1, 0))),
                xspec(lambda n, d: (n, 0, 2 * d)),
                xspec(lambda n, d: (n, 0, 2 * d + 1)),
                xspec(lambda n, d: (n, 0, jnp.minimum(2 * d + 2, D - 1))),
                pl.BlockSpec((Cmid, Cin), lambda n, d: (0, 0)),
                pl.BlockSpec((Cmid, 1), lambda n, d: (0, 0)),
                pl.BlockSpec((9 * 2 * Cmid, 4 * Cmid), lambda n, d: (0, 0)),
                pl.BlockSpec((2 * Cmid, 1), lambda n, d: (0, 0)),
                pl.BlockSpec((2 * Cout, 2 * Cin + 2 * Cmid),
                             lambda n, d: (0, 0)),
                pl.BlockSpec((2 * Cout, 1), lambda n, d: (0, 0)),
            ],
            out_specs=pl.BlockSpec((None, Cout, 2 * HW),
                                   lambda n, d: (n, 0, d)),
            scratch_shapes=[
                pltpu.VMEM((4 * Cmid, PADT), jnp.bfloat16),
            ],
        ),
        compiler_params=pltpu.CompilerParams(
            dimension_semantics=("parallel", "parallel"),
            vmem_limit_bytes=64 * 1024 * 1024),
    )(xb, xb, xb, xb, w1s, b1c, w2s, b2c, wco, bco)
    return out.reshape(N, Cout, D, H, W)


# speedup vs baseline: 2.2655x; 1.2453x over previous
import functools

import jax
import jax.numpy as jnp
from jax.experimental import pallas as pl
from jax.experimental.pallas import tpu as pltpu

_LEAKY = 0.01


def _block_body(H, W, D2, x_m1_ref, x_0_ref, x_p1_ref, x_p2_ref, w1_ref,
                b1_ref, w2s_ref, b2_ref, wsk_ref, w3_ref, bco_ref,
                o_ref, h1pad_ref):
    d2 = pl.program_id(1)
    HW, C = x_0_ref.shape
    PADL = 128
    PADT = h1pad_ref.shape[1]

    xt0 = x_0_ref[...].astype(jnp.bfloat16)
    xt1 = x_p1_ref[...].astype(jnp.bfloat16)
    xcf = [jnp.transpose(x_m1_ref[...].astype(jnp.bfloat16)),
           jnp.transpose(xt0), jnp.transpose(xt1),
           jnp.transpose(x_p2_ref[...].astype(jnp.bfloat16))]

    x4 = jnp.concatenate(xcf, axis=1)
    y4 = jnp.dot(w1_ref[...], x4, preferred_element_type=jnp.float32)
    y4 = y4 + b1_ref[...]
    y4 = jnp.where(y4 >= 0, y4, _LEAKY * y4).astype(jnp.bfloat16)

    h1pad_ref[:, 0:PADL] = jnp.zeros((4 * C, PADL), jnp.bfloat16)
    h1pad_ref[:, PADL + HW:PADT] = jnp.zeros((4 * C, PADT - PADL - HW),
                                             jnp.bfloat16)
    for k in range(4):
        yk = y4[:, k * HW:(k + 1) * HW]
        if k == 0:
            yk = yk * (d2 > 0).astype(jnp.bfloat16)
        elif k == 3:
            yk = yk * (d2 < D2 - 1).astype(jnp.bfloat16)
        h1pad_ref[k * C:(k + 1) * C, PADL:PADL + HW] = yk

    wpos = jax.lax.broadcasted_iota(jnp.int32, (4 * C, HW), 1) % W
    mask_l = wpos != 0
    mask_r = wpos != (W - 1)
    acc = None
    for kh in range(3):
        for kw in range(3):
            j = kh * 3 + kw
            off = PADL + (kh - 1) * W + (kw - 1)
            r = h1pad_ref[:, off:off + HW]
            if kw == 0:
                r = jnp.where(mask_l, r, jnp.bfloat16(0))
            elif kw == 2:
                r = jnp.where(mask_r, r, jnp.bfloat16(0))
            w = w2s_ref[j * 2 * C:(j + 1) * 2 * C, :]
            t = jnp.dot(w, r, preferred_element_type=jnp.float32)
            acc = t if acc is None else acc + t
    h2 = acc + b2_ref[...]
    h2 = jnp.where(h2 >= 0, h2, _LEAKY * h2).astype(jnp.bfloat16)

    dn = (((0,), (0,)), ((), ()))
    b = bco_ref[...]
    y0 = jnp.dot(xt0, wsk_ref[...], preferred_element_type=jnp.float32)
    y0 = y0 + jax.lax.dot_general(h2[0:C, :], w3_ref[...], dn,
                                  preferred_element_type=jnp.float32)
    o_ref[0:HW, :] = jnp.maximum(y0 + b, 0.0)
    y1 = jnp.dot(xt1, wsk_ref[...], preferred_element_type=jnp.float32)
    y1 = y1 + jax.lax.dot_general(h2[C:2 * C, :], w3_ref[...], dn,
                                  preferred_element_type=jnp.float32)
    o_ref[HW:2 * HW, :] = jnp.maximum(y1 + b, 0.0)


def kernel(x, w_skip_mat, s_skip, b_skip, w1_mat, s1, b1, w2_col, s2, b2,
           w3_mat, s3, b3):
    N, Cin, D, H, W = x.shape
    HW = H * W
    D2 = D // 2
    Cmid = w1_mat.shape[1]
    Cout = w_skip_mat.shape[1]
    PADT = 128 + HW + 128

    xs = x.transpose(0, 2, 3, 4, 1).reshape(N, D * HW, Cin)

    w1s = (s1[:, None] * w1_mat.T).astype(jnp.bfloat16)

    w2t = w2_col.reshape(3, 3, 3, Cmid, Cmid)
    w2s = jnp.zeros((9, 2 * Cmid, 4 * Cmid), jnp.float32)
    for kh in range(3):
        for kw in range(3):
            j = kh * 3 + kw
            for kd in range(3):
                wt = s2[:, None] * w2t[kd, kh, kw].T
                w2s = w2s.at[j, 0:Cmid, kd * Cmid:(kd + 1) * Cmid].set(wt)
                w2s = w2s.at[j, Cmid:, (kd + 1) * Cmid:(kd + 2) * Cmid].set(wt)
    w2s = w2s.reshape(9 * 2 * Cmid, 4 * Cmid).astype(jnp.bfloat16)

    wsk = (w_skip_mat * s_skip[None, :]).astype(jnp.bfloat16)
    w3s = (w3_mat * s3[None, :]).astype(jnp.bfloat16)
    b1c = b1.reshape(Cmid, 1)
    b2c = jnp.concatenate([b2, b2]).reshape(2 * Cmid, 1)
    bco = (b_skip + b3).reshape(1, Cout)

    xspec = lambda f: pl.BlockSpec((None, HW, Cin), f)
    out = pl.pallas_call(
        functools.partial(_block_body, H, W, D2),
        out_shape=jax.ShapeDtypeStruct((N, D * HW, Cout), jnp.float32),
        grid_spec=pltpu.PrefetchScalarGridSpec(
            num_scalar_prefetch=0,
            grid=(N, D2),
            in_specs=[
                xspec(lambda n, d: (n, jnp.maximum(2 * d - 1, 0), 0)),
                xspec(lambda n, d: (n, 2 * d, 0)),
                xspec(lambda n, d: (n, 2 * d + 1, 0)),
                xspec(lambda n, d: (n, jnp.minimum(2 * d + 2, D - 1), 0)),
                pl.BlockSpec((Cmid, Cin), lambda n, d: (0, 0)),
                pl.BlockSpec((Cmid, 1), lambda n, d: (0, 0)),
                pl.BlockSpec((9 * 2 * Cmid, 4 * Cmid), lambda n, d: (0, 0)),
                pl.BlockSpec((2 * Cmid, 1), lambda n, d: (0, 0)),
                pl.BlockSpec((Cin, Cout), lambda n, d: (0, 0)),
                pl.BlockSpec((Cmid, Cout), lambda n, d: (0, 0)),
                pl.BlockSpec((1, Cout), lambda n, d: (0, 0)),
            ],
            out_specs=pl.BlockSpec((None, 2 * HW, Cout),
                                   lambda n, d: (n, d, 0)),
            scratch_shapes=[
                pltpu.VMEM((4 * Cmid, PADT), jnp.bfloat16),
            ],
        ),
        compiler_params=pltpu.CompilerParams(
            dimension_semantics=("parallel", "parallel"),
            vmem_limit_bytes=64 * 1024 * 1024),
    )(xs, xs, xs, xs, w1s, b1c, w2s, b2c, wsk, w3s, bco)
    return out.reshape(N, D, H, W, Cout).transpose(0, 4, 1, 2, 3)


# speedup vs baseline: 2.6384x; 1.1646x over previous
import functools

import jax
import jax.numpy as jnp
from jax.experimental import pallas as pl
from jax.experimental.pallas import tpu as pltpu

_LEAKY = 0.01


def _block_body(H, W, D4, x_m1_ref, x_0_ref, x_p1_ref, x_p2_ref, x_p3_ref,
                x_p4_ref, w1_ref, b1_ref, w2s_ref, b2_ref, wsk_ref, w3_ref,
                bco_ref, o_ref, h1pad_ref):
    di = pl.program_id(1)
    HW, C = x_0_ref.shape
    PADL = 128
    PADT = h1pad_ref.shape[1]
    xrefs = [x_m1_ref, x_0_ref, x_p1_ref, x_p2_ref, x_p3_ref, x_p4_ref]

    xt = [r[...].astype(jnp.bfloat16) for r in xrefs]
    xcf = [jnp.transpose(t) for t in xt]

    x6 = jnp.concatenate(xcf, axis=1)
    y6 = jnp.dot(w1_ref[...], x6, preferred_element_type=jnp.float32)
    y6 = y6 + b1_ref[...]
    y6 = jnp.where(y6 >= 0, y6, _LEAKY * y6).astype(jnp.bfloat16)

    h1pad_ref[:, 0:PADL] = jnp.zeros((6 * C, PADL), jnp.bfloat16)
    h1pad_ref[:, PADL + HW:PADT] = jnp.zeros((6 * C, PADT - PADL - HW),
                                             jnp.bfloat16)
    for k in range(6):
        yk = y6[:, k * HW:(k + 1) * HW]
        if k == 0:
            yk = yk * (di > 0).astype(jnp.bfloat16)
        elif k == 5:
            yk = yk * (di < D4 - 1).astype(jnp.bfloat16)
        h1pad_ref[k * C:(k + 1) * C, PADL:PADL + HW] = yk

    wpos = jax.lax.broadcasted_iota(jnp.int32, (6 * C, HW), 1) % W
    mask_l = wpos != 0
    mask_r = wpos != (W - 1)
    acc_a = None
    acc_b = None
    for kh in range(3):
        for kw in range(3):
            j = kh * 3 + kw
            off = PADL + (kh - 1) * W + (kw - 1)
            r = h1pad_ref[:, off:off + HW]
            if kw == 0:
                r = jnp.where(mask_l, r, jnp.bfloat16(0))
            elif kw == 2:
                r = jnp.where(mask_r, r, jnp.bfloat16(0))
            w = w2s_ref[j * 2 * C:(j + 1) * 2 * C, :]
            ta = jnp.dot(w, r[0:4 * C], preferred_element_type=jnp.float32)
            tb = jnp.dot(w, r[2 * C:6 * C],
                         preferred_element_type=jnp.float32)
            acc_a = ta if acc_a is None else acc_a + ta
            acc_b = tb if acc_b is None else acc_b + tb
    b2 = b2_ref[...]
    h2a = acc_a + b2
    h2a = jnp.where(h2a >= 0, h2a, _LEAKY * h2a).astype(jnp.bfloat16)
    h2b = acc_b + b2
    h2b = jnp.where(h2b >= 0, h2b, _LEAKY * h2b).astype(jnp.bfloat16)

    dn = (((0,), (0,)), ((), ()))
    b = bco_ref[...]
    wsk = wsk_ref[...]
    w3 = w3_ref[...]
    h2parts = (h2a[0:C, :], h2a[C:2 * C, :], h2b[0:C, :], h2b[C:2 * C, :])
    for k in range(4):
        y = jnp.dot(xt[k + 1], wsk, preferred_element_type=jnp.float32)
        y = y + jax.lax.dot_general(h2parts[k], w3, dn,
                                    preferred_element_type=jnp.float32)
        o_ref[k * HW:(k + 1) * HW, :] = jnp.maximum(y + b, 0.0)


def kernel(x, w_skip_mat, s_skip, b_skip, w1_mat, s1, b1, w2_col, s2, b2,
           w3_mat, s3, b3):
    N, Cin, D, H, W = x.shape
    HW = H * W
    D4 = D // 4
    Cmid = w1_mat.shape[1]
    Cout = w_skip_mat.shape[1]
    PADT = 128 + HW + 128

    xs = x.transpose(0, 2, 3, 4, 1).reshape(N, D * HW, Cin)

    w1s = (s1[:, None] * w1_mat.T).astype(jnp.bfloat16)

    w2t = w2_col.reshape(3, 3, 3, Cmid, Cmid)
    w2s = jnp.zeros((9, 2 * Cmid, 4 * Cmid), jnp.float32)
    for kh in range(3):
        for kw in range(3):
            j = kh * 3 + kw
            for kd in range(3):
                wt = s2[:, None] * w2t[kd, kh, kw].T
                w2s = w2s.at[j, 0:Cmid, kd * Cmid:(kd + 1) * Cmid].set(wt)
                w2s = w2s.at[j, Cmid:, (kd + 1) * Cmid:(kd + 2) * Cmid].set(wt)
    w2s = w2s.reshape(9 * 2 * Cmid, 4 * Cmid).astype(jnp.bfloat16)

    wsk = (w_skip_mat * s_skip[None, :]).astype(jnp.bfloat16)
    w3s = (w3_mat * s3[None, :]).astype(jnp.bfloat16)
    b1c = b1.reshape(Cmid, 1)
    b2c = jnp.concatenate([b2, b2]).reshape(2 * Cmid, 1)
    bco = (b_skip + b3).reshape(1, Cout)

    xspec = lambda f: pl.BlockSpec((None, HW, Cin), f)
    out = pl.pallas_call(
        functools.partial(_block_body, H, W, D4),
        out_shape=jax.ShapeDtypeStruct((N, D * HW, Cout), jnp.float32),
        grid_spec=pltpu.PrefetchScalarGridSpec(
            num_scalar_prefetch=0,
            grid=(N, D4),
            in_specs=[
                xspec(lambda n, d: (n, jnp.maximum(4 * d - 1, 0), 0)),
                xspec(lambda n, d: (n, 4 * d, 0)),
                xspec(lambda n, d: (n, 4 * d + 1, 0)),
                xspec(lambda n, d: (n, 4 * d + 2, 0)),
                xspec(lambda n, d: (n, 4 * d + 3, 0)),
                xspec(lambda n, d: (n, jnp.minimum(4 * d + 4, D - 1), 0)),
                pl.BlockSpec((Cmid, Cin), lambda n, d: (0, 0)),
                pl.BlockSpec((Cmid, 1), lambda n, d: (0, 0)),
                pl.BlockSpec((9 * 2 * Cmid, 4 * Cmid), lambda n, d: (0, 0)),
                pl.BlockSpec((2 * Cmid, 1), lambda n, d: (0, 0)),
                pl.BlockSpec((Cin, Cout), lambda n, d: (0, 0)),
                pl.BlockSpec((Cmid, Cout), lambda n, d: (0, 0)),
                pl.BlockSpec((1, Cout), lambda n, d: (0, 0)),
            ],
            out_specs=pl.BlockSpec((None, 4 * HW, Cout),
                                   lambda n, d: (n, d, 0)),
            scratch_shapes=[
                pltpu.VMEM((6 * Cmid, PADT), jnp.bfloat16),
            ],
        ),
        compiler_params=pltpu.CompilerParams(
            dimension_semantics=("parallel", "parallel"),
            vmem_limit_bytes=64 * 1024 * 1024),
    )(xs, xs, xs, xs, xs, xs, w1s, b1c, w2s, b2c, wsk, w3s, bco)
    return out.reshape(N, D, H, W, Cout).transpose(0, 4, 1, 2, 3)


# speedup vs baseline: 3.0436x; 1.1535x over previous
import functools

import jax
import jax.numpy as jnp
from jax.experimental import pallas as pl
from jax.experimental.pallas import tpu as pltpu

_LEAKY = 0.01


def _block_body(H, W, NB, xlo_ref, xm_ref, xhi_ref, w1_ref, b1_ref, w2s_ref,
                b2_ref, wsk_ref, w3_ref, bco_ref, o_ref, h1pad_ref):
    di = pl.program_id(1)
    HW, C = xlo_ref.shape
    PADL = 128
    PADT = h1pad_ref.shape[1]

    xm = xm_ref[...].astype(jnp.bfloat16)
    xt = ([xlo_ref[...].astype(jnp.bfloat16)] +
          [xm[k * HW:(k + 1) * HW] for k in range(8)] +
          [xhi_ref[...].astype(jnp.bfloat16)])
    xcf = [jnp.transpose(t) for t in xt]

    x10 = jnp.concatenate(xcf, axis=1)
    y10 = jnp.dot(w1_ref[...], x10, preferred_element_type=jnp.float32)
    y10 = y10 + b1_ref[...]
    y10 = jnp.where(y10 >= 0, y10, _LEAKY * y10).astype(jnp.bfloat16)

    h1pad_ref[:, 0:PADL] = jnp.zeros((10 * C, PADL), jnp.bfloat16)
    h1pad_ref[:, PADL + HW:PADT] = jnp.zeros((10 * C, PADT - PADL - HW),
                                             jnp.bfloat16)
    for k in range(10):
        yk = y10[:, k * HW:(k + 1) * HW]
        if k == 0:
            yk = yk * (di > 0).astype(jnp.bfloat16)
        elif k == 9:
            yk = yk * (di < NB - 1).astype(jnp.bfloat16)
        h1pad_ref[k * C:(k + 1) * C, PADL:PADL + HW] = yk

    wpos = jax.lax.broadcasted_iota(jnp.int32, (6 * C, HW), 1) % W
    mask_l = wpos != 0
    mask_r = wpos != (W - 1)
    b2 = b2_ref[...]
    dn = (((0,), (0,)), ((), ()))
    b = bco_ref[...]
    wsk = wsk_ref[...]
    w3 = w3_ref[...]
    for h in range(2):
        base = h * 4 * C
        acc_a = None
        acc_b = None
        for kh in range(3):
            for kw in range(3):
                j = kh * 3 + kw
                off = PADL + (kh - 1) * W + (kw - 1)
                r = h1pad_ref[base:base + 6 * C, off:off + HW]
                if kw == 0:
                    r = jnp.where(mask_l, r, jnp.bfloat16(0))
                elif kw == 2:
                    r = jnp.where(mask_r, r, jnp.bfloat16(0))
                w = w2s_ref[j * 2 * C:(j + 1) * 2 * C, :]
                ta = jnp.dot(w, r[0:4 * C],
                             preferred_element_type=jnp.float32)
                tb = jnp.dot(w, r[2 * C:6 * C],
                             preferred_element_type=jnp.float32)
                acc_a = ta if acc_a is None else acc_a + ta
                acc_b = tb if acc_b is None else acc_b + tb
        h2a = acc_a + b2
        h2a = jnp.where(h2a >= 0, h2a, _LEAKY * h2a).astype(jnp.bfloat16)
        h2b = acc_b + b2
        h2b = jnp.where(h2b >= 0, h2b, _LEAKY * h2b).astype(jnp.bfloat16)

        h2parts = (h2a[0:C, :], h2a[C:2 * C, :], h2b[0:C, :], h2b[C:2 * C, :])
        for k in range(4):
            kd = h * 4 + k
            y = jnp.dot(xt[kd + 1], wsk, preferred_element_type=jnp.float32)
            y = y + jax.lax.dot_general(h2parts[k], w3, dn,
                                        preferred_element_type=jnp.float32)
            o_ref[kd * HW:(kd + 1) * HW, :] = jnp.maximum(y + b, 0.0)


def kernel(x, w_skip_mat, s_skip, b_skip, w1_mat, s1, b1, w2_col, s2, b2,
           w3_mat, s3, b3):
    N, Cin, D, H, W = x.shape
    HW = H * W
    NB = D // 8
    Cmid = w1_mat.shape[1]
    Cout = w_skip_mat.shape[1]
    PADT = 128 + HW + 128

    xs = x.transpose(0, 2, 3, 4, 1).reshape(N, D * HW, Cin)

    w1s = (s1[:, None] * w1_mat.T).astype(jnp.bfloat16)

    w2t = w2_col.reshape(3, 3, 3, Cmid, Cmid)
    w2s = jnp.zeros((9, 2 * Cmid, 4 * Cmid), jnp.float32)
    for kh in range(3):
        for kw in range(3):
            j = kh * 3 + kw
            for kd in range(3):
                wt = s2[:, None] * w2t[kd, kh, kw].T
                w2s = w2s.at[j, 0:Cmid, kd * Cmid:(kd + 1) * Cmid].set(wt)
                w2s = w2s.at[j, Cmid:, (kd + 1) * Cmid:(kd + 2) * Cmid].set(wt)
    w2s = w2s.reshape(9 * 2 * Cmid, 4 * Cmid).astype(jnp.bfloat16)

    wsk = (w_skip_mat * s_skip[None, :]).astype(jnp.bfloat16)
    w3s = (w3_mat * s3[None, :]).astype(jnp.bfloat16)
    b1c = b1.reshape(Cmid, 1)
    b2c = jnp.concatenate([b2, b2]).reshape(2 * Cmid, 1)
    bco = (b_skip + b3).reshape(1, Cout)

    out = pl.pallas_call(
        functools.partial(_block_body, H, W, NB),
        out_shape=jax.ShapeDtypeStruct((N, D * HW, Cout), jnp.float32),
        grid_spec=pltpu.PrefetchScalarGridSpec(
            num_scalar_prefetch=0,
            grid=(N, NB),
            in_specs=[
                pl.BlockSpec((None, HW, Cin),
                             lambda n, d: (n, jnp.maximum(8 * d - 1, 0), 0)),
                pl.BlockSpec((None, 8 * HW, Cin), lambda n, d: (n, d, 0)),
                pl.BlockSpec((None, HW, Cin),
                             lambda n, d: (n, jnp.minimum(8 * d + 8, D - 1),
                                           0)),
                pl.BlockSpec((Cmid, Cin), lambda n, d: (0, 0)),
                pl.BlockSpec((Cmid, 1), lambda n, d: (0, 0)),
                pl.BlockSpec((9 * 2 * Cmid, 4 * Cmid), lambda n, d: (0, 0)),
                pl.BlockSpec((2 * Cmid, 1), lambda n, d: (0, 0)),
                pl.BlockSpec((Cin, Cout), lambda n, d: (0, 0)),
                pl.BlockSpec((Cmid, Cout), lambda n, d: (0, 0)),
                pl.BlockSpec((1, Cout), lambda n, d: (0, 0)),
            ],
            out_specs=pl.BlockSpec((None, 8 * HW, Cout),
                                   lambda n, d: (n, d, 0)),
            scratch_shapes=[
                pltpu.VMEM((10 * Cmid, PADT), jnp.bfloat16),
            ],
        ),
        compiler_params=pltpu.CompilerParams(
            dimension_semantics=("parallel", "parallel"),
            vmem_limit_bytes=64 * 1024 * 1024),
    )(xs, xs, xs, w1s, b1c, w2s, b2c, wsk, w3s, bco)
    return out.reshape(N, D, H, W, Cout).transpose(0, 4, 1, 2, 3)
